# baseline (speedup 1.0000x reference)
import functools

import jax
import jax.numpy as jnp
from jax.experimental import pallas as pl

B = 32
V = 128
C = 4


def _lin(x, w_ref, b_ref):
    return jax.lax.dot_general(
        x, w_ref[...], (((1,), (1,)), ((), ())),
        preferred_element_type=jnp.float32) + b_ref[...]


def _fused_kernel(edges_ref, nodes_ref, nparams_ref, cond_ref,
                  ne0_W, ne0_b, ne1_W, ne1_b, g1_W, g1_b, g2_W, g2_b,
                  g3a_W, g3a_b, g3b_W, g3b_b, ce0_W, ce0_b, ce1_W, ce1_b,
                  fc0_W, fc0_b, fc1_W, fc1_b, fc2_W, fc2_b, fc3_W, fc3_b,
                  out_ref):
    f32 = jnp.float32
    i_idx = jax.lax.broadcasted_iota(jnp.int32, (V, V * C), 0)
    k_idx = jax.lax.broadcasted_iota(jnp.int32, (V, V * C), 1)
    Msel = jnp.where((k_idx // C == i_idx) & (k_idx % C != 0),
                     f32(1.0), f32(0.0))

    As, degs = [], []
    for b in range(B):
        Eb = edges_ref[b * V * C:(b + 1) * V * C, :]
        Ab = jnp.dot(Msel, Eb, preferred_element_type=f32)
        As.append(Ab)
        degs.append(jnp.sum(Ab, axis=1, keepdims=True) + 1e-8)

    h = _lin(nparams_ref[...], ne0_W, ne0_b)
    h = _lin(h, ne1_W, ne1_b)
    n0 = _lin(jnp.concatenate([nodes_ref[...], h], axis=1), g1_W, g1_b)

    m1_parts = []
    for b in range(B):
        s = slice(b * V, (b + 1) * V)
        m1_parts.append(
            jnp.dot(As[b], n0[s], preferred_element_type=f32) / degs[b])
    m1 = jnp.concatenate(m1_parts, axis=0)

    n1 = _lin(m1, g2_W, g2_b)

    mus = []
    for b in range(B):
        s = slice(b * V, (b + 1) * V)
        degb = degs[b]
        m2 = jnp.dot(As[b], n1[s], preferred_element_type=f32) / degb
        d = m2 - jnp.mean(m2, axis=0, keepdims=True)
        ad = jnp.dot(As[b], d, preferred_element_type=f32)
        ev = ((degb - 1e-8) * d - ad) / degb
        mus.append(jnp.concatenate(
            [jnp.mean(m2, axis=0, keepdims=True),
             jnp.mean(ev, axis=0, keepdims=True)], axis=1))
    mu = jnp.concatenate(mus, axis=0)

    gl = _lin(_lin(mu, g3a_W, g3a_b), g3b_W, g3b_b)
    c = _lin(_lin(cond_ref[...], ce0_W, ce0_b), ce1_W, ce1_b)
    gl = jnp.concatenate([gl, c], axis=1)
    gl = _lin(gl, fc0_W, fc0_b)
    gl = _lin(gl, fc1_W, fc1_b)
    gl = _lin(gl, fc2_W, fc2_b)
    t = gl * fc3_W[...]
    s = jnp.dot(t, jnp.ones((32, V), f32),
                preferred_element_type=f32)
    out_ref[...] = s + fc3_b[0, 0]


@functools.partial(jax.jit, static_argnames=())
def kernel(edges, hidden, nodes, node_params, cond,
           ne0_W, ne0_b, ne1_W, ne1_b, g1_W, g1_b, g2_W, g2_b,
           g3a_W, g3a_b, g3b_W, g3b_b, ce0_W, ce0_b, ce1_W, ce1_b,
           fc0_W, fc0_b, fc1_W, fc1_b, fc2_W, fc2_b, fc3_W, fc3_b):
    del hidden
    edges2d = edges.transpose(0, 1, 3, 2).reshape(B * V * C, V)
    nodes2d = nodes.reshape(B * V, -1)
    nparams2d = node_params.reshape(B * V, -1)
    args = [edges2d, nodes2d, nparams2d, cond,
            ne0_W, ne0_b.reshape(1, -1), ne1_W, ne1_b.reshape(1, -1),
            g1_W, g1_b.reshape(1, -1), g2_W, g2_b.reshape(1, -1),
            g3a_W, g3a_b.reshape(1, -1), g3b_W, g3b_b.reshape(1, -1),
            ce0_W, ce0_b.reshape(1, -1), ce1_W, ce1_b.reshape(1, -1),
            fc0_W, fc0_b.reshape(1, -1), fc1_W, fc1_b.reshape(1, -1),
            fc2_W, fc2_b.reshape(1, -1), fc3_W, fc3_b.reshape(1, -1)]
    out = pl.pallas_call(
        _fused_kernel,
        out_shape=jax.ShapeDtypeStruct((B, V), jnp.float32),
    )(*args)
    return out[:, :1]


# speedup vs baseline: 2.0299x; 2.0299x over previous
import functools

import jax
import jax.numpy as jnp
from jax.experimental import pallas as pl

B = 32
V = 128
C = 4

_F32 = jnp.float32


def _dot(x, y, dims):
    return jax.lax.dot_general(x, y, (dims, ((), ())),
                               preferred_element_type=_F32)


_NN = (((1,), (0,)))
_NT = (((1,), (1,)))
_TT = (((0,), (1,)))
_TN = (((0,), (0,)))


def _fused_kernel(edges_ref, nodesT_ref, nparamsT_ref, condT_ref,
                  ne0_Wt, ne0_b, ne1_W, ne1_b, g1_W, g1_b, g2_Wt, g2_b,
                  g3a_W, g3a_b, g3b_W, g3b_b, ce0_Wt, ce0_b, ce1_W, ce1_b,
                  fc0_Wt, fc0_b, fc1_W, fc1_b, fc2_W, fc2_b, fc3_W, fc3_b,
                  out_ref):
    (ne0_Wt, ne0_b, ne1_W, ne1_b, g1_W, g1_b, g2_Wt, g2_b,
     g3a_W, g3a_b, g3b_W, g3b_b, ce0_Wt, ce0_b, ce1_W, ce1_b,
     fc0_Wt, fc0_b, fc1_W, fc1_b, fc2_W, fc2_b, fc3_W) = (
        r[...] for r in (
            ne0_Wt, ne0_b, ne1_W, ne1_b, g1_W, g1_b, g2_Wt, g2_b,
            g3a_W, g3a_b, g3b_W, g3b_b, ce0_Wt, ce0_b, ce1_W, ce1_b,
            fc0_Wt, fc0_b, fc1_W, fc1_b, fc2_W, fc2_b, fc3_W))
    i_idx = jax.lax.broadcasted_iota(jnp.int32, (V, V * C), 0)
    k_idx = jax.lax.broadcasted_iota(jnp.int32, (V, V * C), 1)
    Msel = jnp.where((k_idx // C == i_idx) & (k_idx % C != 0),
                     _F32(1.0), _F32(0.0))

    W_ne = _dot(ne1_W, ne0_Wt, _NT)
    b_ne = _dot(ne0_b, ne1_W, _NT) + ne1_b
    g1n = g1_W[:, :8]
    g1h = g1_W[:, 8:]
    Wp = _dot(g1h, W_ne, _NN)
    b0 = _dot(b_ne, g1h, _NT) + g1_b

    As, degs, n0s = [], [], []
    for b in range(B):
        Eb = edges_ref[b * V * C:(b + 1) * V * C, :]
        Ab = _dot(Msel, Eb, _NN)
        As.append(Ab)
        degs.append(jnp.sum(Ab, axis=1, keepdims=True) + 1e-8)
        Xn = nodesT_ref[b * 8:(b + 1) * 8, :]
        Xp = nparamsT_ref[b * 16:(b + 1) * 16, :]
        n0s.append(_dot(Xn, g1n, _TT) + _dot(Xp, Wp, _TT) + b0)

    m1_parts = [
        _dot(As[b], n0s[b], _NN) / degs[b] for b in range(B)]
    m1 = jnp.concatenate(m1_parts, axis=0)

    n1 = _dot(m1, g2_Wt, _NN) + g2_b

    mus = []
    for b in range(B):
        s = slice(b * V, (b + 1) * V)
        degb = degs[b]
        m2 = _dot(As[b], n1[s], _NN) / degb
        d = m2 - jnp.mean(m2, axis=0, keepdims=True)
        ad = _dot(As[b], d, _NN)
        ev = ((degb - 1e-8) * d - ad) / degb
        mus.append(jnp.concatenate(
            [jnp.mean(m2, axis=0, keepdims=True),
             jnp.mean(ev, axis=0, keepdims=True)], axis=1))
    mu = jnp.concatenate(mus, axis=0)

    gl = _dot(mu, g3a_W, _NT) + g3a_b
    gl = _dot(gl, g3b_W, _NT) + g3b_b
    c = _dot(condT_ref[...], ce0_Wt, _TN) + ce0_b
    c = _dot(c, ce1_W, _NT) + ce1_b
    gl = jnp.concatenate([gl, c], axis=1)
    gl = _dot(gl, fc0_Wt, _NN) + fc0_b
    gl = _dot(gl, fc1_W, _NT) + fc1_b
    gl = _dot(gl, fc2_W, _NT) + fc2_b
    t = gl * fc3_W
    s = _dot(t, jnp.ones((32, V), _F32), _NN)
    out_ref[...] = s + fc3_b[0, 0]


@functools.partial(jax.jit, static_argnames=())
def kernel(edges, hidden, nodes, node_params, cond,
           ne0_W, ne0_b, ne1_W, ne1_b, g1_W, g1_b, g2_W, g2_b,
           g3a_W, g3a_b, g3b_W, g3b_b, ce0_W, ce0_b, ce1_W, ce1_b,
           fc0_W, fc0_b, fc1_W, fc1_b, fc2_W, fc2_b, fc3_W, fc3_b):
    del hidden
    edges2d = edges.transpose(0, 1, 3, 2).reshape(B * V * C, V)
    nodesT = nodes.transpose(0, 2, 1).reshape(B * 8, V)
    nparamsT = node_params.transpose(0, 2, 1).reshape(B * 16, V)
    args = [edges2d, nodesT, nparamsT, cond.T,
            ne0_W.T, ne0_b.reshape(1, -1), ne1_W, ne1_b.reshape(1, -1),
            g1_W, g1_b.reshape(1, -1), g2_W.T, g2_b.reshape(1, -1),
            g3a_W, g3a_b.reshape(1, -1), g3b_W, g3b_b.reshape(1, -1),
            ce0_W.T, ce0_b.reshape(1, -1), ce1_W, ce1_b.reshape(1, -1),
            fc0_W.T, fc0_b.reshape(1, -1), fc1_W, fc1_b.reshape(1, -1),
            fc2_W, fc2_b.reshape(1, -1), fc3_W, fc3_b.reshape(1, -1)]
    out = pl.pallas_call(
        _fused_kernel,
        out_shape=jax.ShapeDtypeStruct((B, V), jnp.float32),
    )(*args)
    return out[:, :1]
